# baseline (speedup 1.0000x reference)
import functools

import jax
import jax.numpy as jnp
from jax import lax
from jax.experimental import pallas as pl
from jax.experimental.pallas import tpu as pltpu
from jax.experimental.pallas import tpu_sc as plsc

B, S, D = 4, 4096, 1024
NC, NS = 2, 16
NW = NC * NS
S_PER_W = S // NW
CH = 16
NCH = S_PER_W // CH
CHW = CH * D
T = NCH * B


def _add_chunk(x_ref, p_ref):
    def it(k, _):
        base = k * 128
        for u in range(8):
            sl = pl.ds(base + u * 16, 16)
            x_ref[sl] = x_ref[sl] + p_ref[sl]
        return 0
    lax.fori_loop(0, CHW // 128, it, 0)


def _sc_body(x_hbm, p_hbm, o_hbm, x0, x1, p0, p1, sl0, sl1, ss0, ss1, sp0, sp1):
    wid = lax.axis_index("s") * NC + lax.axis_index("c")
    base = wid * (S_PER_W * D)
    xbuf = (x0, x1)
    pbuf = (p0, p1)
    slsem = (sl0, sl1)
    sssem = (ss0, ss1)
    spsem = (sp0, sp1)

    def x_off(t):
        c, b = divmod(t, B)
        return b * (S * D) + base + c * CHW

    def start_load(t):
        return pltpu.async_copy(
            x_hbm.at[pl.ds(x_off(t), CHW)], xbuf[t % 2], slsem[t % 2])

    def start_pos(c):
        return pltpu.async_copy(
            p_hbm.at[pl.ds(base + c * CHW, CHW)], pbuf[c % 2], spsem[c % 2])

    loads = {0: start_load(0)}
    pos_loads = {0: start_pos(0)}
    stores = {}
    for t in range(T):
        bt = t % 2
        if t + 1 < T:
            if t - 1 >= 0:
                stores[t - 1].wait()
            loads[t + 1] = start_load(t + 1)
        c, b = divmod(t, B)
        if b == 0:
            pos_loads[c].wait()
            if c + 1 < NCH:
                pos_loads[c + 1] = start_pos(c + 1)
        loads[t].wait()
        _add_chunk(xbuf[bt], pbuf[c % 2])
        stores[t] = pltpu.async_copy(
            xbuf[bt], o_hbm.at[pl.ds(x_off(t), CHW)], sssem[bt])
    stores[T - 2].wait()
    stores[T - 1].wait()


def kernel(inputs, pos_embedding):
    x = jnp.reshape(inputs, (B * S * D,))
    p = jnp.reshape(pos_embedding, (S * D,))
    mesh = plsc.VectorSubcoreMesh(core_axis_name="c", subcore_axis_name="s")
    run = functools.partial(
        pl.kernel,
        mesh=mesh,
        out_type=jax.ShapeDtypeStruct((B * S * D,), jnp.float32),
        scratch_types=[
            pltpu.VMEM((CHW,), jnp.float32),
            pltpu.VMEM((CHW,), jnp.float32),
            pltpu.VMEM((CHW,), jnp.float32),
            pltpu.VMEM((CHW,), jnp.float32),
            pltpu.SemaphoreType.DMA,
            pltpu.SemaphoreType.DMA,
            pltpu.SemaphoreType.DMA,
            pltpu.SemaphoreType.DMA,
            pltpu.SemaphoreType.DMA,
            pltpu.SemaphoreType.DMA,
        ],
    )(_sc_body)
    out = run(x, p)
    return jnp.reshape(out, (B, S, D))


# speedup vs baseline: 4.7282x; 4.7282x over previous
import jax
import jax.numpy as jnp
from jax.experimental import pallas as pl


def _add_body(x_ref, p_ref, o_ref):
    o_ref[...] = x_ref[...] + p_ref[...]


def kernel(inputs, pos_embedding):
    B, S, D = inputs.shape
    S_BLK = 2048
    grid = (S // S_BLK, B)
    return pl.pallas_call(
        _add_body,
        grid=grid,
        in_specs=[
            pl.BlockSpec((1, S_BLK, D), lambda i, b: (b, i, 0)),
            pl.BlockSpec((1, S_BLK, D), lambda i, b: (0, i, 0)),
        ],
        out_specs=pl.BlockSpec((1, S_BLK, D), lambda i, b: (b, i, 0)),
        out_shape=jax.ShapeDtypeStruct((B, S, D), inputs.dtype),
    )(inputs, pos_embedding)


# speedup vs baseline: 4.7408x; 1.0027x over previous
import jax
import jax.numpy as jnp
from jax.experimental import pallas as pl
from jax.experimental.pallas import tpu as pltpu

B, S, D = 4, 4096, 1024
CR = 512
T = (B * S) // CR
NP = S // CR
NB = 4


def _body(x_hbm, p_hbm, o_hbm, x_v, o_v, p_v, lsem, ssem, psem):
    def xload(i):
        pltpu.make_async_copy(
            x_hbm.at[pl.ds(i * CR, CR), :], x_v.at[i % NB], lsem.at[i % NB]
        ).start()

    def pload(i):
        pltpu.make_async_copy(
            p_hbm.at[pl.ds(i * CR, CR), :], p_v.at[pl.ds(i * CR, CR), :],
            psem.at[i % NB]
        ).start()

    for i in range(NB):
        xload(i)
        pload(i)

    for i in range(T):
        if i + NB < T:
            xload(i + NB)
            if i + NB < NP:
                pload(i + NB)
        pltpu.make_async_copy(
            x_hbm.at[pl.ds(i * CR, CR), :], x_v.at[i % NB], lsem.at[i % NB]
        ).wait()
        if i < NP:
            pltpu.make_async_copy(
                p_hbm.at[pl.ds(i * CR, CR), :], p_v.at[pl.ds(i * CR, CR), :],
                psem.at[i % NB]
            ).wait()
        if i >= NB:
            pltpu.make_async_copy(
                o_v.at[i % NB], o_hbm.at[pl.ds((i - NB) * CR, CR), :],
                ssem.at[i % NB]
            ).wait()
        pr = (i % NP) * CR
        o_v[i % NB] = x_v[i % NB] + p_v[pl.ds(pr, CR), :]
        pltpu.make_async_copy(
            o_v.at[i % NB], o_hbm.at[pl.ds(i * CR, CR), :], ssem.at[i % NB]
        ).start()

    for i in range(T - NB, T):
        pltpu.make_async_copy(
            o_v.at[i % NB], o_hbm.at[pl.ds(i * CR, CR), :], ssem.at[i % NB]
        ).wait()


def kernel(inputs, pos_embedding):
    x = jnp.reshape(inputs, (B * S, D))
    p = jnp.reshape(pos_embedding, (S, D))
    out = pl.pallas_call(
        _body,
        in_specs=[
            pl.BlockSpec(memory_space=pl.ANY),
            pl.BlockSpec(memory_space=pl.ANY),
        ],
        out_specs=pl.BlockSpec(memory_space=pl.ANY),
        out_shape=jax.ShapeDtypeStruct((B * S, D), jnp.float32),
        scratch_shapes=[
            pltpu.VMEM((NB, CR, D), jnp.float32),
            pltpu.VMEM((NB, CR, D), jnp.float32),
            pltpu.VMEM((S, D), jnp.float32),
            pltpu.SemaphoreType.DMA((NB,)),
            pltpu.SemaphoreType.DMA((NB,)),
            pltpu.SemaphoreType.DMA((NB,)),
        ],
    )(x, p)
    return jnp.reshape(out, (B, S, D))


# speedup vs baseline: 4.7439x; 1.0007x over previous
import jax
import jax.numpy as jnp
from jax.experimental import pallas as pl
from jax.experimental.pallas import tpu as pltpu

B, S, D = 4, 4096, 1024
CR = 512
T = (B * S) // CR
NP = S // CR
NB = 4
AH = NB - 1


def _body(x_hbm, p_hbm, o_hbm, x_v, o_v, p_v, lsem, ssem, psem):
    def xload(i):
        return pltpu.make_async_copy(
            x_hbm.at[pl.ds(i * CR, CR), :], x_v.at[i % NB], lsem.at[i % NB])

    def pload(i):
        return pltpu.make_async_copy(
            p_hbm.at[pl.ds(i * CR, CR), :], p_v.at[pl.ds(i * CR, CR), :],
            psem.at[i])

    def store(i):
        return pltpu.make_async_copy(
            o_v.at[i % NB], o_hbm.at[pl.ds(i * CR, CR), :], ssem.at[i % NB])

    for i in range(AH):
        xload(i).start()
        pload(i).start()

    for i in range(T):
        if i + AH < T:
            xload(i + AH).start()
            if i + AH < NP:
                pload(i + AH).start()
        xload(i).wait()
        if i < NP:
            pload(i).wait()
        if i >= NB:
            store(i - NB).wait()
        pr = (i % NP) * CR
        o_v[i % NB] = x_v[i % NB] + p_v[pl.ds(pr, CR), :]
        store(i).start()

    for i in range(T - NB, T):
        store(i).wait()


def kernel(inputs, pos_embedding):
    x = jnp.reshape(inputs, (B * S, D))
    p = jnp.reshape(pos_embedding, (S, D))
    out = pl.pallas_call(
        _body,
        in_specs=[
            pl.BlockSpec(memory_space=pl.ANY),
            pl.BlockSpec(memory_space=pl.ANY),
        ],
        out_specs=pl.BlockSpec(memory_space=pl.ANY),
        out_shape=jax.ShapeDtypeStruct((B * S, D), jnp.float32),
        scratch_shapes=[
            pltpu.VMEM((NB, CR, D), jnp.float32),
            pltpu.VMEM((NB, CR, D), jnp.float32),
            pltpu.VMEM((S, D), jnp.float32),
            pltpu.SemaphoreType.DMA((NB,)),
            pltpu.SemaphoreType.DMA((NB,)),
            pltpu.SemaphoreType.DMA((NP,)),
        ],
    )(x, p)
    return jnp.reshape(out, (B, S, D))


# speedup vs baseline: 4.7469x; 1.0006x over previous
import jax
import jax.numpy as jnp
from jax.experimental import pallas as pl
from jax.experimental.pallas import tpu as pltpu

B, S, D = 4, 4096, 1024
CR = 512
T = (B * S) // CR
NP = S // CR
NB = 6
AH = NB - 1


def _body(x_hbm, p_hbm, o_hbm, x_v, o_v, p_v, lsem, ssem, psem):
    def xload(i):
        return pltpu.make_async_copy(
            x_hbm.at[pl.ds(i * CR, CR), :], x_v.at[i % NB], lsem.at[i % NB])

    def pload(i):
        return pltpu.make_async_copy(
            p_hbm.at[pl.ds(i * CR, CR), :], p_v.at[pl.ds(i * CR, CR), :],
            psem.at[i])

    def store(i):
        return pltpu.make_async_copy(
            o_v.at[i % NB], o_hbm.at[pl.ds(i * CR, CR), :], ssem.at[i % NB])

    for i in range(AH):
        xload(i).start()
        pload(i).start()

    for i in range(T):
        if i + AH < T:
            xload(i + AH).start()
            if i + AH < NP:
                pload(i + AH).start()
        xload(i).wait()
        if i < NP:
            pload(i).wait()
        if i >= NB:
            store(i - NB).wait()
        pr = (i % NP) * CR
        o_v[i % NB] = x_v[i % NB] + p_v[pl.ds(pr, CR), :]
        store(i).start()

    for i in range(T - NB, T):
        store(i).wait()


def kernel(inputs, pos_embedding):
    x = jnp.reshape(inputs, (B * S, D))
    p = jnp.reshape(pos_embedding, (S, D))
    out = pl.pallas_call(
        _body,
        in_specs=[
            pl.BlockSpec(memory_space=pl.ANY),
            pl.BlockSpec(memory_space=pl.ANY),
        ],
        out_specs=pl.BlockSpec(memory_space=pl.ANY),
        out_shape=jax.ShapeDtypeStruct((B * S, D), jnp.float32),
        scratch_shapes=[
            pltpu.VMEM((NB, CR, D), jnp.float32),
            pltpu.VMEM((NB, CR, D), jnp.float32),
            pltpu.VMEM((S, D), jnp.float32),
            pltpu.SemaphoreType.DMA((NB,)),
            pltpu.SemaphoreType.DMA((NB,)),
            pltpu.SemaphoreType.DMA((NP,)),
        ],
    )(x, p)
    return jnp.reshape(out, (B, S, D))
